# baseline (speedup 1.0000x reference)
import functools

import jax
import jax.numpy as jnp
from jax import lax
from jax.experimental import pallas as pl
from jax.experimental.pallas import tpu as pltpu
from jax.experimental.pallas import tpu_sc as plsc

_ROWS, _COLS = 16384, 1024
_NC, _NS, _L = 2, 16, 16
_NW = _NC * _NS

_SC_ROWS = 5120
_TC_ROWS = _ROWS - _SC_ROWS
_TC_BLK = 1024

_ROWS_W = _SC_ROWS // _NW
_CROWS = 32
_NCHUNK = _ROWS_W // _CROWS


@functools.partial(
    pl.kernel,
    mesh=plsc.VectorSubcoreMesh(core_axis_name="c", subcore_axis_name="s"),
    out_type=jax.ShapeDtypeStruct((_NW * _L,), jnp.float32),
    scratch_types=[
        pltpu.VMEM((2, _CROWS, _COLS), jnp.float32),
        pltpu.VMEM((_L,), jnp.float32),
        pltpu.SemaphoreType.DMA,
        pltpu.SemaphoreType.DMA,
    ],
    compiler_params=pltpu.CompilerParams(use_tc_tiling_on_sc=True),
)
def _sc_absmax(x_hbm, out_hbm, buf, part, sem0, sem1):
    wid = lax.axis_index("s") * _NC + lax.axis_index("c")
    row0 = wid * _ROWS_W
    sems = (sem0, sem1)

    handles = [None, None]
    handles[0] = pltpu.async_copy(
        x_hbm.at[pl.ds(row0, _CROWS)], buf.at[0], sems[0])
    accs = tuple(jnp.zeros((_L,), jnp.float32) for _ in range(8))
    for g in range(_NCHUNK):
        b = g % 2
        if g + 1 < _NCHUNK:
            handles[1 - b] = pltpu.async_copy(
                x_hbm.at[pl.ds(row0 + (g + 1) * _CROWS, _CROWS)],
                buf.at[1 - b], sems[1 - b])
        handles[b].wait()

        def rbody(r, a):
            new = list(a)
            for j in range(_COLS // _L):
                v = buf[b, r, pl.ds(j * _L, _L)]
                k = j % 8
                new[k] = jnp.maximum(new[k], jnp.abs(v))
            return tuple(new)

        accs = lax.fori_loop(0, _CROWS, rbody, accs, unroll=2)

    acc = accs[0]
    for k in range(1, 8):
        acc = jnp.maximum(acc, accs[k])
    part[...] = acc
    pltpu.sync_copy(part, out_hbm.at[pl.ds(wid * _L, _L)])


def _tc_reduce_body(x_ref, o_ref):
    i = pl.program_id(0)
    m = jnp.max(jnp.abs(x_ref[...]))

    @pl.when(i == 0)
    def _():
        o_ref[0, 0] = m

    @pl.when(i > 0)
    def _():
        o_ref[0, 0] = jnp.maximum(o_ref[0, 0], m)


def _finalize_body(p_ref, t_ref, o_ref):
    m = jnp.maximum(jnp.max(p_ref[...]), t_ref[0, 0])
    r = 1.0 / m
    bits = lax.bitcast_convert_type(r, jnp.int32)
    e = ((bits >> 23) & 0xFF) - 127
    o_ref[0, 0] = -jnp.clip(e.astype(jnp.float32), -15.0, 15.0)


def kernel(x, _):
    parts_sc = _sc_absmax(x)
    part_tc = pl.pallas_call(
        _tc_reduce_body,
        grid=(_TC_ROWS // _TC_BLK,),
        in_specs=[pl.BlockSpec(
            (_TC_BLK, _COLS),
            lambda i: (i + _SC_ROWS // _TC_BLK, 0))],
        out_specs=pl.BlockSpec(memory_space=pltpu.SMEM),
        out_shape=jax.ShapeDtypeStruct((1, 1), jnp.float32),
    )(x)
    out = pl.pallas_call(
        _finalize_body,
        out_shape=jax.ShapeDtypeStruct((1, 1), jnp.float32),
        out_specs=pl.BlockSpec(memory_space=pltpu.SMEM),
    )(parts_sc.reshape(_NW, _L), part_tc)
    return out[0, 0]


# speedup vs baseline: 1.3900x; 1.3900x over previous
import functools

import jax
import jax.numpy as jnp
from jax import lax
from jax.experimental import pallas as pl
from jax.experimental.pallas import tpu as pltpu
from jax.experimental.pallas import tpu_sc as plsc

_ROWS, _COLS = 16384, 1024
_NC, _NS, _L = 2, 16, 16
_NW = _NC * _NS

_SC_ROWS = 5120
_TC_ROWS = _ROWS - _SC_ROWS
_TC_BLK = 1024

_ROWS_W = _SC_ROWS // _NW
_CROWS = 32
_NCHUNK = _ROWS_W // _CROWS


@functools.partial(
    pl.kernel,
    mesh=plsc.VectorSubcoreMesh(core_axis_name="c", subcore_axis_name="s"),
    out_type=jax.ShapeDtypeStruct((_NW * _L,), jnp.float32),
    scratch_types=[
        pltpu.VMEM((2, _CROWS, _COLS), jnp.float32),
        pltpu.VMEM((_L,), jnp.float32),
        pltpu.SemaphoreType.DMA,
        pltpu.SemaphoreType.DMA,
    ],
    compiler_params=pltpu.CompilerParams(use_tc_tiling_on_sc=True),
)
def _sc_absmax(x_hbm, out_hbm, buf, part, sem0, sem1):
    wid = lax.axis_index("s") * _NC + lax.axis_index("c")
    row0 = wid * _ROWS_W
    sems = (sem0, sem1)

    handles = [None, None]
    handles[0] = pltpu.async_copy(
        x_hbm.at[pl.ds(row0, _CROWS)], buf.at[0], sems[0])
    accs = tuple(jnp.zeros((_L,), jnp.float32) for _ in range(8))
    for g in range(_NCHUNK):
        b = g % 2
        if g + 1 < _NCHUNK:
            handles[1 - b] = pltpu.async_copy(
                x_hbm.at[pl.ds(row0 + (g + 1) * _CROWS, _CROWS)],
                buf.at[1 - b], sems[1 - b])
        handles[b].wait()

        def rbody(r, a):
            new = list(a)
            for j in range(_COLS // _L):
                v = buf[b, r, pl.ds(j * _L, _L)]
                k = j % 8
                new[k] = jnp.maximum(new[k], jnp.abs(v))
            return tuple(new)

        accs = lax.fori_loop(0, _CROWS, rbody, accs)

    acc = accs[0]
    for k in range(1, 8):
        acc = jnp.maximum(acc, accs[k])
    part[...] = acc
    pltpu.sync_copy(part, out_hbm.at[pl.ds(wid * _L, _L)])


def _tc_reduce_body(x_ref, o_ref):
    i = pl.program_id(0)
    m = jnp.max(jnp.abs(x_ref[...]))

    @pl.when(i == 0)
    def _():
        o_ref[0, 0] = m

    @pl.when(i > 0)
    def _():
        o_ref[0, 0] = jnp.maximum(o_ref[0, 0], m)


def _finalize_body(p_ref, t_ref, o_ref):
    m = jnp.maximum(jnp.max(p_ref[...]), t_ref[0, 0])
    r = 1.0 / m
    bits = lax.bitcast_convert_type(r, jnp.int32)
    e = ((bits >> 23) & 0xFF) - 127
    o_ref[0, 0] = -jnp.clip(e.astype(jnp.float32), -15.0, 15.0)


def kernel(x, _):
    parts_sc = _sc_absmax(x)
    part_tc = pl.pallas_call(
        _tc_reduce_body,
        grid=(_TC_ROWS // _TC_BLK,),
        in_specs=[pl.BlockSpec(
            (_TC_BLK, _COLS),
            lambda i: (i + _SC_ROWS // _TC_BLK, 0))],
        out_specs=pl.BlockSpec(memory_space=pltpu.SMEM),
        out_shape=jax.ShapeDtypeStruct((1, 1), jnp.float32),
    )(x)
    out = pl.pallas_call(
        _finalize_body,
        out_shape=jax.ShapeDtypeStruct((1, 1), jnp.float32),
        out_specs=pl.BlockSpec(memory_space=pltpu.SMEM),
    )(parts_sc, part_tc)
    return out[0, 0]


# speedup vs baseline: 1.3907x; 1.0006x over previous
import functools

import jax
import jax.numpy as jnp
from jax import lax
from jax.experimental import pallas as pl
from jax.experimental.pallas import tpu as pltpu
from jax.experimental.pallas import tpu_sc as plsc

_ROWS, _COLS = 16384, 1024
_NC, _NS, _L = 2, 16, 16
_NW = _NC * _NS

_SC_ROWS = 5120
_TC_ROWS = _ROWS - _SC_ROWS
_TC_BLK = 1024

_ROWS_W = _SC_ROWS // _NW
_CROWS = 32
_NCHUNK = _ROWS_W // _CROWS


@functools.partial(
    pl.kernel,
    mesh=plsc.VectorSubcoreMesh(core_axis_name="c", subcore_axis_name="s"),
    out_type=jax.ShapeDtypeStruct((_NW * _L,), jnp.float32),
    scratch_types=[
        pltpu.VMEM((2, _CROWS, _COLS), jnp.float32),
        pltpu.VMEM((_L,), jnp.float32),
        pltpu.SemaphoreType.DMA,
        pltpu.SemaphoreType.DMA,
    ],
    compiler_params=pltpu.CompilerParams(use_tc_tiling_on_sc=True),
)
def _sc_absmax(x_hbm, out_hbm, buf, part, sem0, sem1):
    wid = lax.axis_index("s") * _NC + lax.axis_index("c")
    row0 = wid * _ROWS_W
    sems = (sem0, sem1)

    handles = [None, None]
    handles[0] = pltpu.async_copy(
        x_hbm.at[pl.ds(row0, _CROWS)], buf.at[0], sems[0])
    accs = tuple(jnp.zeros((_L,), jnp.float32) for _ in range(8))
    for g in range(_NCHUNK):
        b = g % 2
        if g + 1 < _NCHUNK:
            handles[1 - b] = pltpu.async_copy(
                x_hbm.at[pl.ds(row0 + (g + 1) * _CROWS, _CROWS)],
                buf.at[1 - b], sems[1 - b])
        handles[b].wait()

        @plsc.parallel_loop(0, _CROWS, carry=accs)
        def accs(r, a):
            new = list(a)
            for j in range(_COLS // _L):
                v = buf[b, r, pl.ds(j * _L, _L)]
                k = j % 8
                new[k] = jnp.maximum(new[k], jnp.abs(v))
            return tuple(new)

    acc = accs[0]
    for k in range(1, 8):
        acc = jnp.maximum(acc, accs[k])
    part[...] = acc
    pltpu.sync_copy(part, out_hbm.at[pl.ds(wid * _L, _L)])


def _tc_reduce_body(x_ref, o_ref):
    i = pl.program_id(0)
    m = jnp.max(jnp.abs(x_ref[...]))

    @pl.when(i == 0)
    def _():
        o_ref[0, 0] = m

    @pl.when(i > 0)
    def _():
        o_ref[0, 0] = jnp.maximum(o_ref[0, 0], m)


def _finalize_body(p_ref, t_ref, o_ref):
    m = jnp.maximum(jnp.max(p_ref[...]), t_ref[0, 0])
    r = 1.0 / m
    bits = lax.bitcast_convert_type(r, jnp.int32)
    e = ((bits >> 23) & 0xFF) - 127
    o_ref[0, 0] = -jnp.clip(e.astype(jnp.float32), -15.0, 15.0)


def kernel(x, _):
    parts_sc = _sc_absmax(x)
    part_tc = pl.pallas_call(
        _tc_reduce_body,
        grid=(_TC_ROWS // _TC_BLK,),
        in_specs=[pl.BlockSpec(
            (_TC_BLK, _COLS),
            lambda i: (i + _SC_ROWS // _TC_BLK, 0))],
        out_specs=pl.BlockSpec(memory_space=pltpu.SMEM),
        out_shape=jax.ShapeDtypeStruct((1, 1), jnp.float32),
    )(x)
    out = pl.pallas_call(
        _finalize_body,
        out_shape=jax.ShapeDtypeStruct((1, 1), jnp.float32),
        out_specs=pl.BlockSpec(memory_space=pltpu.SMEM),
    )(parts_sc, part_tc)
    return out[0, 0]
